# baseline (speedup 1.0000x reference)
import jax
import jax.numpy as jnp
from jax import lax
from jax.experimental import pallas as pl
from jax.experimental.pallas import tpu as pltpu
from jax.experimental.pallas import tpu_sc as plsc

NUM_EMBEDDINGS = 100000
OUTPUT_DIM = 64
BATCH = 16384
N_FIELDS = 26

NC = 2
NS = 16
L = 16
NW = NC * NS

TOTAL = BATCH * N_FIELDS
B_PER_W = TOTAL // NW
BATCH_PER_W = BATCH // NW
BPC = 4
CHUNK = BPC * N_FIELDS
GPS = 4
BPS = BPC * GPS
SUPER = CHUNK * GPS
N_SUPER = B_PER_W // SUPER
N_PAIR = N_SUPER // 2


def _embed_kernel(x_hbm, table_hbm, out_hbm, idx_v, bufs, gsems, ssems):
    wid = lax.axis_index("s") * NC + lax.axis_index("c")
    base = wid * B_PER_W
    bbase = wid * BATCH_PER_W

    pltpu.sync_copy(x_hbm.at[pl.ds(base, B_PER_W)], idx_v)

    def shift_body(i, _):
        s = pl.ds(i * L, L)
        idx_v[s] = idx_v[s] + 1
        return ()

    lax.fori_loop(0, B_PER_W // L, shift_body, (), unroll=8)

    def fire_gathers(s, b):
        for i in range(GPS):
            pltpu.make_async_copy(
                table_hbm.at[idx_v.at[pl.ds(s * SUPER + i * CHUNK, CHUNK)]],
                bufs.at[b, pl.ds(i * CHUNK, CHUNK)],
                gsems.at[b],
            ).start()

    def wait_gathers(s, b):
        for i in range(GPS):
            pltpu.make_async_copy(
                table_hbm.at[idx_v.at[pl.ds(s * SUPER + i * CHUNK, CHUNK)]],
                bufs.at[b, pl.ds(i * CHUNK, CHUNK)],
                gsems.at[b],
            ).wait()

    def fire_stores(s, b):
        for j in range(BPS):
            pltpu.make_async_copy(
                bufs.at[b, pl.ds(j * N_FIELDS, N_FIELDS)],
                out_hbm.at[bbase + s * BPS + j],
                ssems.at[b],
            ).start()

    def wait_stores(s, b):
        for j in range(BPS):
            pltpu.make_async_copy(
                bufs.at[b, pl.ds(j * N_FIELDS, N_FIELDS)],
                out_hbm.at[bbase + s * BPS + j],
                ssems.at[b],
            ).wait()

    fire_gathers(0, 0)

    def pair_body(g, _):
        s0 = 2 * g
        s1 = s0 + 1

        @pl.when(g > 0)
        def _():
            wait_stores(s1 - 2, 1)

        fire_gathers(s1, 1)
        wait_gathers(s0, 0)
        fire_stores(s0, 0)
        wait_gathers(s1, 1)

        @pl.when(g + 1 < N_PAIR)
        def _():
            wait_stores(s0, 0)
            fire_gathers(s0 + 2, 0)

        fire_stores(s1, 1)
        return ()

    lax.fori_loop(0, N_PAIR, pair_body, ())

    wait_stores(N_SUPER - 2, 0)
    wait_stores(N_SUPER - 1, 1)


@jax.jit
def kernel(x, table):
    x_flat = x.reshape(TOTAL)
    mesh = plsc.VectorSubcoreMesh(
        core_axis_name="c", subcore_axis_name="s", num_cores=NC, num_subcores=NS
    )
    out = pl.kernel(
        _embed_kernel,
        out_type=jax.ShapeDtypeStruct((BATCH, N_FIELDS, OUTPUT_DIM), jnp.float32),
        mesh=mesh,
        scratch_types=[
            pltpu.VMEM((B_PER_W,), jnp.int32),
            pltpu.VMEM((2, SUPER, OUTPUT_DIM), jnp.float32),
            pltpu.SemaphoreType.DMA((2,)),
            pltpu.SemaphoreType.DMA((2,)),
        ],
        compiler_params=pltpu.CompilerParams(use_tc_tiling_on_sc=False),
    )(x_flat, table)
    return out
